# baseline (speedup 1.0000x reference)
import functools
import jax
import jax.numpy as jnp
import numpy as np
from jax import lax
from jax.experimental import pallas as pl
from jax.experimental.pallas import tpu as pltpu
from jax.experimental.pallas import tpu_sc as plsc

NUM_ITERATIONS = 5
THETA_ALPHA = 8.0
THETA_BETA = 0.25
THETA_GAMMA = 2.0
B, H, W, C = 8, 224, 224, 21

GY = int(np.ceil((H - 1) / THETA_ALPHA)) + 1
GX = int(np.ceil((W - 1) / THETA_ALPHA)) + 1
GC = int(np.ceil(1.0 / THETA_BETA)) + 1
NCELLS = GY * GX * GC * GC * GC
SCRAP = NCELLS
NTILES = 16
STRIPE = 6576
GR = NTILES * STRIPE
HWPIX = H * W
HWPAD = 51200
PPT = HWPAD // NTILES
NCHUNK = PPT // 128

_sc_mesh = plsc.VectorSubcoreMesh(core_axis_name="c", subcore_axis_name="s")


@functools.partial(
    pl.kernel,
    out_type=jax.ShapeDtypeStruct((B, 2, GR, 16), jnp.float32),
    mesh=_sc_mesh,
    scratch_types=[
        pltpu.VMEM((NCHUNK, 128), jnp.int32),
        pltpu.VMEM((128, 16), jnp.float32),
        pltpu.VMEM_SHARED((GR, 16), jnp.float32),
    ],
    compiler_params=pltpu.CompilerParams(use_tc_tiling_on_sc=False),
)
def _sc_splat(vals_hbm, idx_hbm, zeros_hbm, grid_hbm, idx_v, vals_v, grid_sp):
    cid = lax.axis_index("c")
    sid = lax.axis_index("s")
    for b in range(B):
        pltpu.sync_copy(zeros_hbm.at[pl.ds(sid * STRIPE, STRIPE)],
                        grid_sp.at[pl.ds(sid * STRIPE, STRIPE)])
        plsc.subcore_barrier()
        pltpu.sync_copy(idx_hbm.at[b, sid], idx_v)

        def chunk(j, carry):
            pltpu.sync_copy(vals_hbm.at[cid, b, pl.ds(sid * PPT + j * 128, 128)],
                            vals_v)
            pltpu.sync_copy(vals_v, grid_sp.at[idx_v.at[j]], add=True)
            return carry

        lax.fori_loop(0, NCHUNK, chunk, 0)
        plsc.subcore_barrier()
        pltpu.sync_copy(grid_sp.at[pl.ds(sid * STRIPE, STRIPE)],
                        grid_hbm.at[b, cid, pl.ds(sid * STRIPE, STRIPE)])


@functools.partial(
    pl.kernel,
    out_type=jax.ShapeDtypeStruct((2, B, HWPAD, 16), jnp.float32),
    mesh=_sc_mesh,
    scratch_types=[
        pltpu.VMEM((NCHUNK, 128), jnp.int32),
        pltpu.VMEM((128, 16), jnp.float32),
        pltpu.SemaphoreType.DMA,
    ],
    compiler_params=pltpu.CompilerParams(use_tc_tiling_on_sc=False),
)
def _sc_slice(grid_hbm, idx_hbm, out_hbm, idx_v, rows_v, sem):
    cid = lax.axis_index("c")
    sid = lax.axis_index("s")
    for b in range(B):
        pltpu.sync_copy(idx_hbm.at[b, sid], idx_v)

        def chunk(j, carry):
            pltpu.async_copy(grid_hbm.at[b, cid].at[idx_v.at[j]],
                             rows_v, sem).wait()
            pltpu.sync_copy(rows_v,
                            out_hbm.at[cid, b, pl.ds(sid * PPT + j * 128, 128)])
            return carry

        lax.fori_loop(0, NCHUNK, chunk, 0)


def _blur_axis(x, kernel, axis):
    r = (kernel.shape[0] - 1) // 2
    pads = [(0, 0)] * x.ndim
    pads[axis] = (r, r)
    xp = jnp.pad(x, pads)
    n = x.shape[axis]
    out = jnp.zeros_like(x)
    for i in range(kernel.shape[0]):
        sl = jax.lax.dynamic_slice_in_dim(xp, i, n, axis)
        out = out + kernel[i] * sl
    return out


def _gaussian_filter_spatial(Q, sigma):
    radius = int(np.ceil(3.0 * sigma))
    offs = np.arange(-radius, radius + 1, dtype=np.float64)
    k = np.exp(-(offs ** 2) / (2.0 * sigma * sigma))
    k = jnp.asarray((k / k.sum()).astype(np.float32))
    ones = jnp.ones(Q.shape[:-1] + (1,), Q.dtype)
    x = jnp.concatenate([Q, ones], axis=-1)
    x = _blur_axis(x, k, 1)
    x = _blur_axis(x, k, 2)
    norm = jnp.maximum(x[..., -1:], 1e-6)
    return x[..., :-1] / norm


def _bilateral_indices(I):
    ys = lax.broadcasted_iota(jnp.float32, (H, W), 0)
    xs = lax.broadcasted_iota(jnp.float32, (H, W), 1)
    fy = jnp.clip(jnp.round(ys / THETA_ALPHA).astype(jnp.int32), 0, GY - 1)
    fx = jnp.clip(jnp.round(xs / THETA_ALPHA).astype(jnp.int32), 0, GX - 1)
    fr = jnp.clip(jnp.round(I[..., 0] / THETA_BETA).astype(jnp.int32), 0, GC - 1)
    fg = jnp.clip(jnp.round(I[..., 1] / THETA_BETA).astype(jnp.int32), 0, GC - 1)
    fb = jnp.clip(jnp.round(I[..., 2] / THETA_BETA).astype(jnp.int32), 0, GC - 1)
    lin = ((((fy[None] * GX + fx[None]) * GC + fr) * GC + fg) * GC + fb)
    lin = lin.reshape(B, HWPIX)
    lin = jnp.pad(lin, ((0, 0), (0, HWPAD - HWPIX)), constant_values=SCRAP)
    return lin.reshape(B, NTILES, NCHUNK, 128)


def _bilateral_batch(S, idx, zeros_col):
    ones = jnp.ones((B, H, W, 1), jnp.float32)
    zpad = jnp.zeros((B, H, W, 32 - (C + 1)), jnp.float32)
    v = jnp.concatenate([S, ones, zpad], axis=-1).reshape(B, HWPIX, 2, 16)
    v = jnp.pad(v, ((0, 0), (0, HWPAD - HWPIX), (0, 0), (0, 0)))
    v = v.transpose(2, 0, 1, 3)

    grid = _sc_splat(v, idx, zeros_col)

    g = grid[:, :, :NCELLS, :].reshape(B, 2, GY, GX, GC, GC, GC, 16)
    k3 = jnp.asarray(np.array([0.25, 0.5, 0.25], np.float32))
    for ax in range(2, 7):
        g = _blur_axis(g, k3, ax)
    g = g.reshape(B, 2, NCELLS, 16)
    g = jnp.pad(g, ((0, 0), (0, 0), (0, GR - NCELLS), (0, 0)))

    sl = _sc_slice(g, idx)
    sl = sl[:, :, :HWPIX, :].transpose(1, 2, 0, 3).reshape(B, H, W, 32)
    return sl[..., :C] / jnp.maximum(sl[..., C:C + 1], 1e-6)


def _combine_kernel(q0_ref, q1_ref, u_ref, k0_ref, k1_ref, comp_ref, out_ref):
    q = q0_ref[...] * k0_ref[...] + q1_ref[...] * k1_ref[...]
    q = jax.lax.dot_general(q.reshape(-1, C), comp_ref[...],
                            (((1,), (0,)), ((), ())),
                            preferred_element_type=jnp.float32)
    out_ref[...] = u_ref[...] - q.reshape(out_ref.shape)


def _combine(Q0, Q1, U, K0, K1, comp):
    return pl.pallas_call(
        _combine_kernel,
        out_shape=jax.ShapeDtypeStruct((B, H, W, C), jnp.float32),
        grid=(B, 7),
        in_specs=[
            pl.BlockSpec((1, 32, W, C), lambda b, h: (b, h, 0, 0)),
            pl.BlockSpec((1, 32, W, C), lambda b, h: (b, h, 0, 0)),
            pl.BlockSpec((1, 32, W, C), lambda b, h: (b, h, 0, 0)),
            pl.BlockSpec((C,), lambda b, h: (0,)),
            pl.BlockSpec((C,), lambda b, h: (0,)),
            pl.BlockSpec((C, C), lambda b, h: (0, 0)),
        ],
        out_specs=pl.BlockSpec((1, 32, W, C), lambda b, h: (b, h, 0, 0)),
    )(Q0, Q1, U, K0, K1, comp)


def kernel(I, U, K0_weights, K1_weights, compatibility_matrix):
    idx = _bilateral_indices(I)
    zeros_col = jnp.zeros((GR, 16), jnp.float32)
    Q = U
    for _ in range(NUM_ITERATIONS):
        S = jax.nn.softmax(Q, axis=-1)
        Q0 = _gaussian_filter_spatial(S, THETA_GAMMA)
        Q1 = _bilateral_batch(S, idx, zeros_col)
        Q = _combine(Q0, Q1, U, K0_weights, K1_weights, compatibility_matrix)
    return Q


# speedup vs baseline: 2.0863x; 2.0863x over previous
import functools
import jax
import jax.numpy as jnp
import numpy as np
from jax import lax
from jax.experimental import pallas as pl
from jax.experimental.pallas import tpu as pltpu
from jax.experimental.pallas import tpu_sc as plsc

NUM_ITERATIONS = 5
THETA_ALPHA = 8.0
THETA_BETA = 0.25
THETA_GAMMA = 2.0
B, H, W, C = 8, 224, 224, 21

GY = int(np.ceil((H - 1) / THETA_ALPHA)) + 1
GX = int(np.ceil((W - 1) / THETA_ALPHA)) + 1
GC = int(np.ceil(1.0 / THETA_BETA)) + 1
NCELLS = GY * GX * GC * GC * GC
SCRAP = NCELLS
NTILES = 16
STRIPE = 6576
GR = NTILES * STRIPE
HWPIX = H * W
HWPAD = 51200
PPT = HWPAD // NTILES
NCHUNK = PPT // 128
NWORK = 2 * NTILES
SCHUNK = 13
HWPAD2 = NWORK * SCHUNK * 128
PPW = HWPAD2 // NWORK

_sc_mesh = plsc.VectorSubcoreMesh(core_axis_name="c", subcore_axis_name="s")


@functools.partial(
    pl.kernel,
    out_type=jax.ShapeDtypeStruct((B, GR, 32), jnp.float32),
    mesh=_sc_mesh,
    scratch_types=[
        pltpu.VMEM((NCHUNK, 128), jnp.int32),
        pltpu.VMEM((128, 16), jnp.float32),
        pltpu.VMEM_SHARED((GR, 16), jnp.float32),
    ],
    compiler_params=pltpu.CompilerParams(use_tc_tiling_on_sc=False),
)
def _sc_splat(vals_hbm, idx_hbm, zeros_hbm, grid_hbm, idx_v, vals_v, grid_sp):
    cid = lax.axis_index("c")
    sid = lax.axis_index("s")
    for b in range(B):
        pltpu.sync_copy(zeros_hbm.at[pl.ds(sid * STRIPE, STRIPE)],
                        grid_sp.at[pl.ds(sid * STRIPE, STRIPE)])
        plsc.subcore_barrier()
        pltpu.sync_copy(idx_hbm.at[b, sid], idx_v)

        def chunk(j, carry):
            pltpu.sync_copy(
                vals_hbm.at[b, pl.ds(sid * PPT + j * 128, 128),
                            pl.ds(cid * 16, 16)], vals_v)
            pltpu.sync_copy(vals_v, grid_sp.at[idx_v.at[j]], add=True)
            return carry

        lax.fori_loop(0, NCHUNK, chunk, 0)
        plsc.subcore_barrier()
        pltpu.sync_copy(grid_sp.at[pl.ds(sid * STRIPE, STRIPE)],
                        grid_hbm.at[b, pl.ds(sid * STRIPE, STRIPE),
                                    pl.ds(cid * 16, 16)])


@functools.partial(
    pl.kernel,
    out_type=jax.ShapeDtypeStruct((B, HWPAD2, 32), jnp.float32),
    mesh=_sc_mesh,
    scratch_types=[
        pltpu.VMEM((SCHUNK, 128), jnp.int32),
        pltpu.VMEM((128, 32), jnp.float32),
        pltpu.SemaphoreType.DMA,
    ],
    compiler_params=pltpu.CompilerParams(use_tc_tiling_on_sc=False),
)
def _sc_slice(grid_hbm, idx_hbm, out_hbm, idx_v, rows_v, sem):
    wid = lax.axis_index("s") * 2 + lax.axis_index("c")
    for b in range(B):
        pltpu.sync_copy(idx_hbm.at[b, wid], idx_v)

        def chunk(j, carry):
            pltpu.async_copy(grid_hbm.at[b].at[idx_v.at[j]],
                             rows_v, sem).wait()
            pltpu.sync_copy(rows_v,
                            out_hbm.at[b, pl.ds(wid * PPW + j * 128, 128)])
            return carry

        lax.fori_loop(0, SCHUNK, chunk, 0)


def _blur_axis(x, kernel, axis):
    r = (kernel.shape[0] - 1) // 2
    pads = [(0, 0)] * x.ndim
    pads[axis] = (r, r)
    xp = jnp.pad(x, pads)
    n = x.shape[axis]
    out = jnp.zeros_like(x)
    for i in range(kernel.shape[0]):
        sl = jax.lax.dynamic_slice_in_dim(xp, i, n, axis)
        out = out + kernel[i] * sl
    return out


def _gaussian_filter_spatial(Q, sigma):
    radius = int(np.ceil(3.0 * sigma))
    offs = np.arange(-radius, radius + 1, dtype=np.float64)
    k = np.exp(-(offs ** 2) / (2.0 * sigma * sigma))
    k = jnp.asarray((k / k.sum()).astype(np.float32))
    ones = jnp.ones(Q.shape[:-1] + (1,), Q.dtype)
    x = jnp.concatenate([Q, ones], axis=-1)
    x = _blur_axis(x, k, 1)
    x = _blur_axis(x, k, 2)
    norm = jnp.maximum(x[..., -1:], 1e-6)
    return x[..., :-1] / norm


def _bilateral_indices(I):
    ys = lax.broadcasted_iota(jnp.float32, (H, W), 0)
    xs = lax.broadcasted_iota(jnp.float32, (H, W), 1)
    fy = jnp.clip(jnp.round(ys / THETA_ALPHA).astype(jnp.int32), 0, GY - 1)
    fx = jnp.clip(jnp.round(xs / THETA_ALPHA).astype(jnp.int32), 0, GX - 1)
    fr = jnp.clip(jnp.round(I[..., 0] / THETA_BETA).astype(jnp.int32), 0, GC - 1)
    fg = jnp.clip(jnp.round(I[..., 1] / THETA_BETA).astype(jnp.int32), 0, GC - 1)
    fb = jnp.clip(jnp.round(I[..., 2] / THETA_BETA).astype(jnp.int32), 0, GC - 1)
    lin = ((((fy[None] * GX + fx[None]) * GC + fr) * GC + fg) * GC + fb)
    lin = lin.reshape(B, HWPIX)
    idx = jnp.pad(lin, ((0, 0), (0, HWPAD - HWPIX)), constant_values=SCRAP)
    idx2 = jnp.pad(lin, ((0, 0), (0, HWPAD2 - HWPIX)), constant_values=SCRAP)
    return (idx.reshape(B, NTILES, NCHUNK, 128),
            idx2.reshape(B, NWORK, SCHUNK, 128))


def _bilateral_batch(S, idx, idx2, zeros_col):
    ones = jnp.ones((B, H, W, 1), jnp.float32)
    zpad = jnp.zeros((B, H, W, 32 - (C + 1)), jnp.float32)
    v = jnp.concatenate([S, ones, zpad], axis=-1).reshape(B, HWPIX, 32)
    v = jnp.pad(v, ((0, 0), (0, HWPAD - HWPIX), (0, 0)))

    grid = _sc_splat(v, idx, zeros_col)

    g = grid[:, :NCELLS, :].reshape(B, GY, GX, GC, GC, GC, 32)
    k3 = jnp.asarray(np.array([0.25, 0.5, 0.25], np.float32))
    for ax in range(1, 6):
        g = _blur_axis(g, k3, ax)
    g = g.reshape(B, NCELLS, 32)
    g = jnp.pad(g, ((0, 0), (0, GR - NCELLS), (0, 0)))

    sl = _sc_slice(g, idx2)
    sl = sl[:, :HWPIX, :].reshape(B, H, W, 32)
    return sl[..., :C] / jnp.maximum(sl[..., C:C + 1], 1e-6)


def kernel(I, U, K0_weights, K1_weights, compatibility_matrix):
    idx, idx2 = _bilateral_indices(I)
    zeros_col = jnp.zeros((GR, 16), jnp.float32)
    Q = U
    for _ in range(NUM_ITERATIONS):
        S = jax.nn.softmax(Q, axis=-1)
        Q0 = _gaussian_filter_spatial(S, THETA_GAMMA)
        Q1 = _bilateral_batch(S, idx, idx2, zeros_col)
        Q = Q0 * K0_weights + Q1 * K1_weights
        Q = jnp.tensordot(Q, compatibility_matrix, axes=[[3], [0]])
        Q = U - Q
    return Q


# speedup vs baseline: 2.0967x; 1.0049x over previous
import functools
import jax
import jax.numpy as jnp
import numpy as np
from jax import lax
from jax.experimental import pallas as pl
from jax.experimental.pallas import tpu as pltpu
from jax.experimental.pallas import tpu_sc as plsc

NUM_ITERATIONS = 5
THETA_ALPHA = 8.0
THETA_BETA = 0.25
THETA_GAMMA = 2.0
B, H, W, C = 8, 224, 224, 21

GY = int(np.ceil((H - 1) / THETA_ALPHA)) + 1
GX = int(np.ceil((W - 1) / THETA_ALPHA)) + 1
GC = int(np.ceil(1.0 / THETA_BETA)) + 1
NCELLS = GY * GX * GC * GC * GC
SCRAP = NCELLS
NTILES = 16
STRIPE = 6576
GR = NTILES * STRIPE
HWPIX = H * W
HWPAD = 51200
PPT = HWPAD // NTILES
NCHUNK = PPT // 128
NWORK = 2 * NTILES
SCHUNK = 13
HWPAD2 = NWORK * SCHUNK * 128
PPW = HWPAD2 // NWORK

_sc_mesh = plsc.VectorSubcoreMesh(core_axis_name="c", subcore_axis_name="s")


@functools.partial(
    pl.kernel,
    out_type=jax.ShapeDtypeStruct((B, GR, 32), jnp.float32),
    mesh=_sc_mesh,
    scratch_types=[
        pltpu.VMEM((NCHUNK, 128), jnp.int32),
        pltpu.VMEM((128, 16), jnp.float32),
        pltpu.VMEM((128, 16), jnp.float32),
        pltpu.VMEM_SHARED((GR, 16), jnp.float32),
        pltpu.SemaphoreType.DMA,
        pltpu.SemaphoreType.DMA,
    ],
    compiler_params=pltpu.CompilerParams(use_tc_tiling_on_sc=False),
)
def _sc_splat(vals_hbm, idx_hbm, zeros_hbm, grid_hbm, idx_v, vals_a, vals_b,
              grid_sp, sem_a, sem_b):
    cid = lax.axis_index("c")
    sid = lax.axis_index("s")
    bufs = (vals_a, vals_b)
    sems = (sem_a, sem_b)
    for b in range(B):
        pltpu.sync_copy(zeros_hbm.at[pl.ds(sid * STRIPE, STRIPE)],
                        grid_sp.at[pl.ds(sid * STRIPE, STRIPE)])
        plsc.subcore_barrier()
        pltpu.sync_copy(idx_hbm.at[b, sid], idx_v)

        def fetch(j):
            return pltpu.async_copy(
                vals_hbm.at[b, pl.ds(sid * PPT + j * 128, 128),
                            pl.ds(cid * 16, 16)], bufs[j % 2], sems[j % 2])

        pending = fetch(0)
        for j in range(NCHUNK):
            nxt = fetch(j + 1) if j + 1 < NCHUNK else None
            pending.wait()
            pltpu.sync_copy(bufs[j % 2], grid_sp.at[idx_v.at[j]], add=True)
            pending = nxt
        plsc.subcore_barrier()
        pltpu.sync_copy(grid_sp.at[pl.ds(sid * STRIPE, STRIPE)],
                        grid_hbm.at[b, pl.ds(sid * STRIPE, STRIPE),
                                    pl.ds(cid * 16, 16)])


@functools.partial(
    pl.kernel,
    out_type=jax.ShapeDtypeStruct((B, HWPAD2, 32), jnp.float32),
    mesh=_sc_mesh,
    scratch_types=[
        pltpu.VMEM((SCHUNK, 128), jnp.int32),
        pltpu.VMEM((128, 32), jnp.float32),
        pltpu.VMEM((128, 32), jnp.float32),
        pltpu.SemaphoreType.DMA,
        pltpu.SemaphoreType.DMA,
    ],
    compiler_params=pltpu.CompilerParams(use_tc_tiling_on_sc=False),
)
def _sc_slice(grid_hbm, idx_hbm, out_hbm, rows_i, rows_a, rows_b, sem_a, sem_b):
    wid = lax.axis_index("s") * 2 + lax.axis_index("c")
    bufs = (rows_a, rows_b)
    sems = (sem_a, sem_b)
    for b in range(B):
        pltpu.sync_copy(idx_hbm.at[b, wid], rows_i)

        def fetch(j):
            return pltpu.async_copy(grid_hbm.at[b].at[rows_i.at[j]],
                                    bufs[j % 2], sems[j % 2])

        pending = fetch(0)
        for j in range(SCHUNK):
            nxt = fetch(j + 1) if j + 1 < SCHUNK else None
            pending.wait()
            pltpu.sync_copy(bufs[j % 2],
                            out_hbm.at[b, pl.ds(wid * PPW + j * 128, 128)])
            pending = nxt


def _blur_axis(x, kernel, axis):
    r = (kernel.shape[0] - 1) // 2
    pads = [(0, 0)] * x.ndim
    pads[axis] = (r, r)
    xp = jnp.pad(x, pads)
    n = x.shape[axis]
    out = jnp.zeros_like(x)
    for i in range(kernel.shape[0]):
        sl = jax.lax.dynamic_slice_in_dim(xp, i, n, axis)
        out = out + kernel[i] * sl
    return out


def _gaussian_filter_spatial(Q, sigma):
    radius = int(np.ceil(3.0 * sigma))
    offs = np.arange(-radius, radius + 1, dtype=np.float64)
    k = np.exp(-(offs ** 2) / (2.0 * sigma * sigma))
    k = jnp.asarray((k / k.sum()).astype(np.float32))
    ones = jnp.ones(Q.shape[:-1] + (1,), Q.dtype)
    x = jnp.concatenate([Q, ones], axis=-1)
    x = _blur_axis(x, k, 1)
    x = _blur_axis(x, k, 2)
    norm = jnp.maximum(x[..., -1:], 1e-6)
    return x[..., :-1] / norm


def _bilateral_indices(I):
    ys = lax.broadcasted_iota(jnp.float32, (H, W), 0)
    xs = lax.broadcasted_iota(jnp.float32, (H, W), 1)
    fy = jnp.clip(jnp.round(ys / THETA_ALPHA).astype(jnp.int32), 0, GY - 1)
    fx = jnp.clip(jnp.round(xs / THETA_ALPHA).astype(jnp.int32), 0, GX - 1)
    fr = jnp.clip(jnp.round(I[..., 0] / THETA_BETA).astype(jnp.int32), 0, GC - 1)
    fg = jnp.clip(jnp.round(I[..., 1] / THETA_BETA).astype(jnp.int32), 0, GC - 1)
    fb = jnp.clip(jnp.round(I[..., 2] / THETA_BETA).astype(jnp.int32), 0, GC - 1)
    lin = ((((fy[None] * GX + fx[None]) * GC + fr) * GC + fg) * GC + fb)
    lin = lin.reshape(B, HWPIX)
    idx = jnp.pad(lin, ((0, 0), (0, HWPAD - HWPIX)), constant_values=SCRAP)
    idx2 = jnp.pad(lin, ((0, 0), (0, HWPAD2 - HWPIX)), constant_values=SCRAP)
    return (idx.reshape(B, NTILES, NCHUNK, 128),
            idx2.reshape(B, NWORK, SCHUNK, 128))


def _bilateral_batch(S, idx, idx2, zeros_col):
    ones = jnp.ones((B, H, W, 1), jnp.float32)
    zpad = jnp.zeros((B, H, W, 32 - (C + 1)), jnp.float32)
    v = jnp.concatenate([S, ones, zpad], axis=-1).reshape(B, HWPIX, 32)
    v = jnp.pad(v, ((0, 0), (0, HWPAD - HWPIX), (0, 0)))

    grid = _sc_splat(v, idx, zeros_col)

    g = grid[:, :NCELLS, :].reshape(B, GY, GX, GC, GC, GC, 32)
    k3 = jnp.asarray(np.array([0.25, 0.5, 0.25], np.float32))
    for ax in range(1, 6):
        g = _blur_axis(g, k3, ax)
    g = g.reshape(B, NCELLS, 32)
    g = jnp.pad(g, ((0, 0), (0, GR - NCELLS), (0, 0)))

    sl = _sc_slice(g, idx2)
    sl = sl[:, :HWPIX, :].reshape(B, H, W, 32)
    return sl[..., :C] / jnp.maximum(sl[..., C:C + 1], 1e-6)


def kernel(I, U, K0_weights, K1_weights, compatibility_matrix):
    idx, idx2 = _bilateral_indices(I)
    zeros_col = jnp.zeros((GR, 16), jnp.float32)
    Q = U
    for _ in range(NUM_ITERATIONS):
        S = jax.nn.softmax(Q, axis=-1)
        Q0 = _gaussian_filter_spatial(S, THETA_GAMMA)
        Q1 = _bilateral_batch(S, idx, idx2, zeros_col)
        Q = Q0 * K0_weights + Q1 * K1_weights
        Q = jnp.tensordot(Q, compatibility_matrix, axes=[[3], [0]])
        Q = U - Q
    return Q
